# baseline (speedup 1.0000x reference)
import functools

import jax
import jax.numpy as jnp
from jax import lax
from jax.experimental import pallas as pl
from jax.experimental.pallas import tpu as pltpu
from jax.experimental.pallas import tpu_sc as plsc

_NC = 2
_NS = 16
_NW = _NC * _NS
_C = 16
_NBUF = 4
_NSH = 2


def _make_lookup(B, V, D):
    b_per_w = B // _NW
    n_chunks = b_per_w // _C
    assert n_chunks % _NBUF == 0 and _NBUF % _NSH == 0
    mesh = plsc.VectorSubcoreMesh(core_axis_name="c", subcore_axis_name="s")

    @functools.partial(
        pl.kernel,
        out_type=jax.ShapeDtypeStruct((B, D), jnp.float32),
        mesh=mesh,
        scratch_types=[
            pltpu.VMEM((b_per_w,), jnp.int32),
            [pltpu.VMEM((_C, D), jnp.float32) for _ in range(_NBUF)],
            pltpu.VMEM_SHARED((_NS, _NSH, _C, D), jnp.float32),
            [pltpu.SemaphoreType.DMA for _ in range(_NBUF)],
            [pltpu.SemaphoreType.DMA for _ in range(_NSH)],
        ],
    )
    def k(table_hbm, idx_hbm, out_hbm, idx_v, vbufs, shared, gsems, wsems):
        cid = lax.axis_index("c")
        sid = lax.axis_index("s")
        wid = sid * _NC + cid
        base = wid * b_per_w
        pltpu.sync_copy(idx_hbm.at[pl.ds(base, b_per_w)], idx_v)

        def start_gather(c, b):
            pltpu.async_copy(
                table_hbm.at[idx_v.at[pl.ds(c * _C, _C)]], vbufs[b], gsems[b]
            )

        def wait_gather(b):
            pltpu.make_async_copy(
                table_hbm.at[pl.ds(0, _C)], vbufs[b], gsems[b]
            ).wait()

        def start_write(c, s):
            pltpu.async_copy(
                shared.at[sid, s], out_hbm.at[pl.ds(base + c * _C, _C)],
                wsems[s],
            )

        def wait_write(s):
            pltpu.make_async_copy(
                shared.at[sid, s], out_hbm.at[pl.ds(base, _C)], wsems[s]
            ).wait()

        start_gather(0, 0)
        start_gather(1, 1)

        def body(g4, carry):
            for b in range(_NBUF):
                c = g4 * _NBUF + b
                s = b % _NSH

                @pl.when(c >= _NSH)
                def _():
                    wait_write(s)

                wait_gather(b)
                pltpu.sync_copy(vbufs[b], shared.at[sid, s])
                start_write(c, s)

                @pl.when(c + 2 < n_chunks)
                def _():
                    start_gather(c + 2, (b + 2) % _NBUF)
            return carry

        lax.fori_loop(0, n_chunks // _NBUF, body, 0)
        wait_write((n_chunks - 2) % _NSH)
        wait_write((n_chunks - 1) % _NSH)

    return k


def kernel(position_ids, table):
    batch, seq = position_ids.shape
    V, D = table.shape
    flat_ids = position_ids.reshape(-1).astype(jnp.int32)
    out = _make_lookup(flat_ids.shape[0], V, D)(table, flat_ids)
    return out.reshape(batch, seq, D)
